# baseline (speedup 1.0000x reference)
import jax
import jax.numpy as jnp
import numpy as np
from jax.experimental import pallas as pl
from jax.experimental.pallas import tpu as pltpu

H = 16777216
NG = 64
COLS = 1024
ROWS = H // COLS
STRIDE = 266305
IDX = [STRIDE * i for i in range(NG)]
ROW = [v // COLS for v in IDX]
COL = [v % COLS for v in IDX]
NCHUNK = 8
CHROWS = ROWS // NCHUNK


def _body(x_hbm, gw_ref, sc_ref, cols_ref, lc_ref, out_hbm,
          grows, patched, sem_big, sem_g, sem_w):
    big = [
        pltpu.make_async_copy(
            x_hbm.at[pl.ds(k * CHROWS, CHROWS), :],
            out_hbm.at[pl.ds(k * CHROWS, CHROWS), :],
            sem_big,
        )
        for k in range(NCHUNK)
    ]
    for cp in big:
        cp.start()

    gth = [
        pltpu.make_async_copy(
            x_hbm.at[pl.ds(ROW[i], 1), :],
            grows.at[pl.ds(i, 1), :],
            sem_g,
        )
        for i in range(NG)
    ]
    for cp in gth:
        cp.start()
    for cp in gth:
        cp.wait()

    gmat = grows[...]
    lane = jax.lax.broadcasted_iota(jnp.int32, (NG, COLS), 1)
    hit = lane == cols_ref[...]
    vals = jnp.sum(jnp.where(hit, gmat, 0.0), axis=1, keepdims=True)
    gated = vals * gw_ref[...] * jnp.tanh(sc_ref[...])

    mean = jnp.sum(gated) / NG
    var = jnp.sum((gated - mean) ** 2) / (NG - 1)
    sig = jax.lax.rem(var, jnp.float32(1e-4)) * 1e-12

    prow = jnp.where(hit, gated, gmat)

    ls = prow[0:1, 0:22]
    mp = lc_ref[...] * (1.0 + sig * 1e9)
    t = jnp.dot(ls, mp, preferred_element_type=jnp.float32)

    patched[...] = prow
    patched[0:1, 0:22] = t

    for cp in big:
        cp.wait()

    wrt = [
        pltpu.make_async_copy(
            patched.at[pl.ds(i, 1), :],
            out_hbm.at[pl.ds(ROW[i], 1), :],
            sem_w,
        )
        for i in range(NG)
    ]
    for cp in wrt:
        cp.start()
    for cp in wrt:
        cp.wait()


def kernel(x, gate_weights, sacred_combinations, aether_gates, letter_combinations):
    del aether_gates
    x2 = x.reshape(ROWS, COLS)
    gw2 = gate_weights.reshape(NG, 1)
    sc2 = sacred_combinations.reshape(NG, 1)
    cols2 = jnp.asarray(np.array(COL, dtype=np.int32).reshape(NG, 1))

    out = pl.pallas_call(
        _body,
        in_specs=[
            pl.BlockSpec(memory_space=pltpu.MemorySpace.HBM),
            pl.BlockSpec(memory_space=pltpu.MemorySpace.VMEM),
            pl.BlockSpec(memory_space=pltpu.MemorySpace.VMEM),
            pl.BlockSpec(memory_space=pltpu.MemorySpace.VMEM),
            pl.BlockSpec(memory_space=pltpu.MemorySpace.VMEM),
        ],
        out_specs=pl.BlockSpec(memory_space=pltpu.MemorySpace.HBM),
        out_shape=jax.ShapeDtypeStruct((ROWS, COLS), jnp.float32),
        scratch_shapes=[
            pltpu.VMEM((NG, COLS), jnp.float32),
            pltpu.VMEM((NG, COLS), jnp.float32),
            pltpu.SemaphoreType.DMA,
            pltpu.SemaphoreType.DMA,
            pltpu.SemaphoreType.DMA,
        ],
    )(x2, gw2, sc2, cols2, letter_combinations)
    return out.reshape(H)


# speedup vs baseline: 46.7835x; 46.7835x over previous
import jax
import jax.numpy as jnp
import numpy as np
from jax.experimental import pallas as pl
from jax.experimental.pallas import tpu as pltpu

H = 16777216
NG = 64
STRIDE = 266305
IDX = [STRIDE * i for i in range(NG)]
SPAN = 128
BASE = [(v // SPAN) * SPAN for v in IDX]
COL = [v % SPAN for v in IDX]
NCH = 16
CHE = H // NCH
NB = 4
K = 2


def _body(x_hbm, gw_ref, sc_ref, col_ref, lc_ref, out_hbm,
          buf, g2d, pw, sems_i, sems_o, sem_g, sem_w):
    gth = [
        pltpu.make_async_copy(
            x_hbm.at[pl.ds(BASE[i], SPAN)], g2d.at[i], sem_g)
        for i in range(NG)
    ]
    for cp in gth:
        cp.start()

    ic = [pltpu.make_async_copy(
            x_hbm.at[pl.ds(i * CHE, CHE)],
            buf.at[pl.ds((i % NB) * CHE, CHE)],
            sems_i.at[i % NB]) for i in range(NCH)]
    oc = [pltpu.make_async_copy(
            buf.at[pl.ds((i % NB) * CHE, CHE)],
            out_hbm.at[pl.ds(i * CHE, CHE)],
            sems_o.at[i % NB]) for i in range(NCH)]
    for i in range(NCH):
        if i >= NB:
            oc[i - NB].wait()
        ic[i].start()
        j = i - K
        if j >= 0:
            ic[j].wait()
            oc[j].start()
    for j in range(NCH - K, NCH):
        ic[j].wait()
        oc[j].start()

    for cp in gth:
        cp.wait()
    gm = g2d[...]
    lane = jax.lax.broadcasted_iota(jnp.int32, (NG, SPAN), 1)
    hit = lane == col_ref[...]
    vals = jnp.sum(jnp.where(hit, gm, 0.0), axis=1, keepdims=True)
    gated = vals * gw_ref[...] * jnp.tanh(sc_ref[...])
    mean = jnp.sum(gated) / NG
    var = jnp.sum((gated - mean) ** 2) / (NG - 1)
    sig = jax.lax.rem(var, jnp.float32(1e-4)) * 1e-12

    rows = jax.lax.broadcasted_iota(jnp.int32, (NG, 1), 0)
    g0 = jnp.sum(jnp.where(rows == 0, gated, 0.0))
    l22 = jax.lax.broadcasted_iota(jnp.int32, (1, 22), 1)
    ls = jnp.where(l22 == 0, g0, gm[0:1, 0:22])
    mp = lc_ref[...] * (1.0 + sig * 1e9)
    t = jnp.dot(ls, mp, preferred_element_type=jnp.float32)

    pw[...] = jnp.where(hit, gated, gm)
    pw[0:1, :] = jnp.concatenate([t, gm[0:1, 22:]], axis=1)

    for j in range(NCH - NB, NCH):
        oc[j].wait()
    wrt = [
        pltpu.make_async_copy(
            pw.at[i], out_hbm.at[pl.ds(BASE[i], SPAN)], sem_w)
        for i in range(NG)
    ]
    for cp in wrt:
        cp.start()
    for cp in wrt:
        cp.wait()


def kernel(x, gate_weights, sacred_combinations, aether_gates, letter_combinations):
    del aether_gates
    gw2 = gate_weights.reshape(NG, 1)
    sc2 = sacred_combinations.reshape(NG, 1)
    col2 = jnp.asarray(np.array(COL, dtype=np.int32).reshape(NG, 1))

    out = pl.pallas_call(
        _body,
        in_specs=[
            pl.BlockSpec(memory_space=pltpu.MemorySpace.HBM),
            pl.BlockSpec(memory_space=pltpu.MemorySpace.VMEM),
            pl.BlockSpec(memory_space=pltpu.MemorySpace.VMEM),
            pl.BlockSpec(memory_space=pltpu.MemorySpace.VMEM),
            pl.BlockSpec(memory_space=pltpu.MemorySpace.VMEM),
        ],
        out_specs=pl.BlockSpec(memory_space=pltpu.MemorySpace.HBM),
        out_shape=jax.ShapeDtypeStruct((H,), jnp.float32),
        scratch_shapes=[
            pltpu.VMEM((NB * CHE,), jnp.float32),
            pltpu.VMEM((NG, SPAN), jnp.float32),
            pltpu.VMEM((NG, SPAN), jnp.float32),
            pltpu.SemaphoreType.DMA((NB,)),
            pltpu.SemaphoreType.DMA((NB,)),
            pltpu.SemaphoreType.DMA,
            pltpu.SemaphoreType.DMA,
        ],
    )(x, gw2, sc2, col2, letter_combinations)
    return out
